# Initial kernel scaffold; baseline (speedup 1.0000x reference)
import jax
import jax.numpy as jnp
from jax.experimental import pallas as pl


def kernel(inputs_0, inputs_1, W1, b1, W2, b2):
    raise NotImplementedError("write your pallas kernel here")


# speedup vs baseline: 13.7647x; 13.7647x over previous
import dataclasses
import functools

import jax
import jax.numpy as jnp
from jax import lax
from jax.experimental import pallas as pl
from jax.experimental.pallas import tpu as pltpu
from jax.experimental.pallas import tpu_sc as plsc

B, N1, N2 = 16, 1024, 4096
C1, C2 = 256, 128
F1, F2 = 256, 256
EPS = 1e-7

T2 = 256
NT2 = N2 // T2
SUB = 32
SPB = T2 // SUB
LANES = 16
TM = 512


def _three_nn_body(xyz1_ref, xyz2t_ref, idx_ref, w_ref):
    b = pl.program_id(0)
    xyz1 = xyz1_ref[0]
    q = xyz2t_ref[0]
    dx = xyz1[:, 0:1] - q[0:1, :]
    dy = xyz1[:, 1:2] - q[1:2, :]
    dz = xyz1[:, 2:3] - q[2:3, :]
    d = (dx * dx + dy * dy) + dz * dz
    iota = lax.broadcasted_iota(jnp.int32, (N1, T2), 0)
    inf = jnp.float32(jnp.inf)
    sel, rcp = [], []
    for _ in range(3):
        cur = d
        for prev in sel:
            cur = jnp.where(iota == prev, inf, cur)
        m = jnp.min(cur, axis=0, keepdims=True)
        cand = jnp.where(cur == m, iota, N1)
        ik = jnp.min(cand, axis=0, keepdims=True)
        sel.append(ik)
        rcp.append(1.0 / jnp.maximum(m, EPS))
    norm = rcp[0] + rcp[1] + rcp[2]
    idx_ref[0, 0] = jnp.concatenate([s + b * N1 for s in sel], axis=0)
    w_ref[0, 0] = jnp.concatenate([r / norm for r in rcp], axis=0)


def _three_nn(xyz1, xyz2t):
    return pl.pallas_call(
        _three_nn_body,
        grid=(B, NT2),
        in_specs=[
            pl.BlockSpec((1, N1, 3), lambda b, t: (b, 0, 0)),
            pl.BlockSpec((1, 3, T2), lambda b, t: (b, 0, t)),
        ],
        out_specs=[
            pl.BlockSpec((1, 1, 3, T2), lambda b, t: (b, t, 0, 0)),
            pl.BlockSpec((1, 1, 3, T2), lambda b, t: (b, t, 0, 0)),
        ],
        out_shape=[
            jax.ShapeDtypeStruct((B, NT2, 3, T2), jnp.int32),
            jax.ShapeDtypeStruct((B, NT2, 3, T2), jnp.float32),
        ],
    )(xyz1, xyz2t)


def _sc_interp(table, idx3, w3):
    mesh = plsc.VectorSubcoreMesh(core_axis_name="c", subcore_axis_name="s")
    nstep = (B * N2) // SUB
    cp = pltpu.CompilerParams()
    if "needs_layout_passes" in pltpu.CompilerParams.__dataclass_fields__:
        cp = dataclasses.replace(cp, needs_layout_passes=False)

    @functools.partial(
        pl.kernel,
        out_type=jax.ShapeDtypeStruct((B * N2, C1), jnp.float32),
        mesh=mesh,
        compiler_params=cp,
        scratch_types=[
            pltpu.VMEM((SUB, C1), jnp.float32),
            pltpu.VMEM((SUB, C1), jnp.float32),
            pltpu.VMEM((SUB, C1), jnp.float32),
        ],
    )
    def run(table_hbm, idx_hbm, w_hbm, out_hbm, r0, r1, r2):
        rows = (r0, r1, r2)

        def body(idx_vm, w_vm, out_vm):
            for k in range(3):
                pltpu.sync_copy(table_hbm.at[idx_vm.at[0, k]], rows[k])

            @pl.loop(0, SUB)
            def _(qv):
                z16 = jnp.zeros((LANES,), jnp.int32)
                qi = jnp.full((LANES,), qv, jnp.int32)
                w = [
                    plsc.load_gather(
                        w_vm, [z16, jnp.full((LANES,), k, jnp.int32), qi]
                    )
                    for k in range(3)
                ]
                for c in range(C1 // LANES):
                    cs = pl.ds(c * LANES, LANES)
                    acc = w[0] * rows[0][qv, cs]
                    acc += w[1] * rows[1][qv, cs]
                    acc += w[2] * rows[2][qv, cs]
                    out_vm[qv, cs] = acc

        pltpu.emit_pipeline(
            body,
            grid=(nstep,),
            in_specs=[
                pl.BlockSpec((1, 3, SUB), lambda i: (i, 0, 0)),
                pl.BlockSpec((1, 3, SUB), lambda i: (i, 0, 0)),
            ],
            out_specs=[pl.BlockSpec((SUB, C1), lambda i: (i, 0))],
            core_axis_name=("c", "s"),
            dimension_semantics=(pltpu.PARALLEL,),
        )(idx_hbm, w_hbm, out_hbm)

    return run(table, idx3, w3)


def _mlp_body(x_ref, p_ref, w1a_ref, w1b_ref, b1_ref, w2_ref, b2_ref, o_ref):
    h = jnp.dot(x_ref[...], w1a_ref[...], preferred_element_type=jnp.float32)
    h += jnp.dot(p_ref[...], w1b_ref[...], preferred_element_type=jnp.float32)
    h = jnp.maximum(h + b1_ref[...], 0.0)
    o = jnp.dot(h, w2_ref[...], preferred_element_type=jnp.float32)
    o_ref[...] = jnp.maximum(o + b2_ref[...], 0.0)


def _mlp(x, p, w1a, w1b, b1, w2, b2):
    nrow = B * N2
    return pl.pallas_call(
        _mlp_body,
        grid=(nrow // TM,),
        in_specs=[
            pl.BlockSpec((TM, C1), lambda i: (i, 0)),
            pl.BlockSpec((TM, C2), lambda i: (i, 0)),
            pl.BlockSpec((C1, F1), lambda i: (0, 0)),
            pl.BlockSpec((C2, F1), lambda i: (0, 0)),
            pl.BlockSpec((1, F1), lambda i: (0, 0)),
            pl.BlockSpec((F1, F2), lambda i: (0, 0)),
            pl.BlockSpec((1, F2), lambda i: (0, 0)),
        ],
        out_specs=pl.BlockSpec((TM, F2), lambda i: (i, 0)),
        out_shape=jax.ShapeDtypeStruct((nrow, F2), jnp.float32),
    )(x, p, w1a, w1b, b1, w2, b2)


def kernel(inputs_0, inputs_1, W1, b1, W2, b2):
    xyz1 = inputs_0[:, :, 0:3]
    points1 = inputs_0[:, :, 3:]
    xyz2 = inputs_1[:, :, 0:3]
    points2 = inputs_1[:, :, 3:]

    xyz2t = jnp.transpose(xyz2, (0, 2, 1))
    idx3, w3 = _three_nn(xyz1, xyz2t)

    table = points1.reshape(B * N1, C1)
    nstep = (B * N2) // SUB

    def _steps(a):
        return (a.reshape(B * NT2, 3, SPB, SUB)
                 .transpose(0, 2, 1, 3)
                 .reshape(nstep, 3, SUB))

    interp = _sc_interp(table, _steps(idx3), _steps(w3))

    h = _mlp(
        interp,
        points2.reshape(B * N2, C2),
        W1[:C1],
        W1[C1:],
        b1.reshape(1, F1),
        W2,
        b2.reshape(1, F2),
    )
    return h.reshape(B, N2, F2), xyz2
